# initial kernel scaffold (unmeasured)
import jax
import jax.numpy as jnp
from jax import lax
from jax.experimental import pallas as pl
from jax.experimental.pallas import tpu as pltpu

N_DEV = 8
SQ = 1024
SKV_SHARD = 1024
HQ = 8
DH = 128
DM = HQ * DH
SCALE = 0.08838834764831843
NEG = -1e9
GLOB_ROWS = 32
BAND_LO = 896
N_SHARED = GLOB_ROWS + (SQ - BAND_LO)

CHILDREN = {0: (4, 3, 1), 4: (7, 5), 3: (2,), 7: (6,)}
PARENT = {1: 0, 3: 0, 4: 0, 5: 4, 7: 4, 2: 3, 6: 7}


def kernel(x, Wq, K_ext, V_ext, Wo):
    x2 = x.reshape(SQ, DM)
    K2 = K_ext.reshape(SKV_SHARD, DM)
    V2 = V_ext.reshape(SKV_SHARD, DM)

    def body(x_ref, wq_ref, k_ref, v_ref, wo_ref, out_ref,
             qbuf, bias, part_n, part_md, send_n, send_md,
             gath_n, gath_md, ctx_bf, bcast,
             gsend_sems, grecvn_sems, grecvmd_sems, bsend_sems, brecv_sem):
        my = lax.axis_index("i")

        barrier = pltpu.get_barrier_semaphore()
        for s in range(N_DEV):
            pl.semaphore_signal(barrier, inc=1, device_id=(s,),
                                device_id_type=pl.DeviceIdType.MESH)
        pl.semaphore_wait(barrier, N_DEV)

        q = jnp.dot(x_ref[...].astype(jnp.bfloat16),
                    wq_ref[...].astype(jnp.bfloat16),
                    preferred_element_type=jnp.float32)
        qbuf[...] = q.astype(jnp.bfloat16)

        rows = lax.broadcasted_iota(jnp.int32, (SQ, SKV_SHARD), 0)
        cols = lax.broadcasted_iota(jnp.int32, (SQ, SKV_SHARD), 1)
        ki = cols + my * SKV_SHARD
        keep = (jnp.abs(rows - ki) <= 128) | (ki < 32) | (rows < 32)
        bias[...] = jnp.where(keep, 0.0, NEG).astype(jnp.float32)

        for h in range(HQ):
            c0, c1 = h * DH, (h + 1) * DH
            kh = k_ref[:, c0:c1].astype(jnp.bfloat16)
            s = lax.dot_general(qbuf[:, c0:c1], kh,
                                (((1,), (1,)), ((), ())),
                                preferred_element_type=jnp.float32)
            s = s * SCALE + bias[...]
            m = jnp.max(s, axis=1, keepdims=True)
            w = jnp.exp(s - m)
            d = jnp.sum(w, axis=1, keepdims=True)
            nh = jnp.dot(w.astype(jnp.bfloat16), v_ref[:, c0:c1].astype(jnp.bfloat16),
                         preferred_element_type=jnp.float32)
            part_n[:, c0:c1] = nh
            part_md[:, h:h + 1] = m
            part_md[:, HQ + h:HQ + h + 1] = d

        send_n[0:GLOB_ROWS, :] = part_n[0:GLOB_ROWS, :]
        send_n[GLOB_ROWS:N_SHARED, :] = part_n[BAND_LO:SQ, :]
        send_md[0:GLOB_ROWS, :] = part_md[0:GLOB_ROWS, :]
        send_md[GLOB_ROWS:N_SHARED, :] = part_md[BAND_LO:SQ, :]

        @pl.when(my == 0)
        def _():
            gath_n[0] = send_n[...]
            gath_md[0] = send_md[...]

        for s_idx in range(1, N_DEV):
            @pl.when(my == s_idx)
            def _(s_idx=s_idx):
                r1 = pltpu.make_async_remote_copy(
                    src_ref=send_n, dst_ref=gath_n.at[s_idx],
                    send_sem=gsend_sems.at[0], recv_sem=grecvn_sems.at[s_idx],
                    device_id=(0,), device_id_type=pl.DeviceIdType.MESH)
                r2 = pltpu.make_async_remote_copy(
                    src_ref=send_md, dst_ref=gath_md.at[s_idx],
                    send_sem=gsend_sems.at[1], recv_sem=grecvmd_sems.at[s_idx],
                    device_id=(0,), device_id_type=pl.DeviceIdType.MESH)
                r1.start()
                r2.start()
                r1.wait_send()
                r2.wait_send()

        @pl.when(my == 0)
        def _():
            for s_idx in range(1, N_DEV):
                rn = pltpu.make_async_remote_copy(
                    src_ref=gath_n.at[s_idx], dst_ref=gath_n.at[s_idx],
                    send_sem=gsend_sems.at[0], recv_sem=grecvn_sems.at[s_idx],
                    device_id=(0,), device_id_type=pl.DeviceIdType.MESH)
                rn.wait_recv()
                rmd = pltpu.make_async_remote_copy(
                    src_ref=gath_md.at[s_idx], dst_ref=gath_md.at[s_idx],
                    send_sem=gsend_sems.at[1], recv_sem=grecvmd_sems.at[s_idx],
                    device_id=(0,), device_id_type=pl.DeviceIdType.MESH)
                rmd.wait_recv()

            for h in range(HQ):
                c0, c1 = h * DH, (h + 1) * DH
                ctx_bf[:, c0:c1] = (
                    part_n[:, c0:c1] / part_md[:, HQ + h:HQ + h + 1]
                ).astype(jnp.bfloat16)

            m_all = gath_md[0, :, 0:HQ]
            for s_idx in range(1, N_DEV):
                m_all = jnp.maximum(m_all, gath_md[s_idx, :, 0:HQ])
            scales = []
            den = jnp.zeros((N_SHARED, HQ), jnp.float32)
            for s_idx in range(N_DEV):
                c = jnp.exp(gath_md[s_idx, :, 0:HQ] - m_all)
                scales.append(c)
                den = den + gath_md[s_idx, :, HQ:2 * HQ] * c
            for h in range(HQ):
                c0, c1 = h * DH, (h + 1) * DH
                num = gath_n[0, :, c0:c1] * scales[0][:, h:h + 1]
                for s_idx in range(1, N_DEV):
                    num = num + gath_n[s_idx, :, c0:c1] * scales[s_idx][:, h:h + 1]
                ch = (num / den[:, h:h + 1]).astype(jnp.bfloat16)
                ctx_bf[0:GLOB_ROWS, c0:c1] = ch[0:GLOB_ROWS, :]
                ctx_bf[BAND_LO:SQ, c0:c1] = ch[GLOB_ROWS:N_SHARED, :]

            o = jnp.dot(ctx_bf[...], wo_ref[...].astype(jnp.bfloat16),
                        preferred_element_type=jnp.float32)
            out_ref[...] = o
            bcast[...] = o.astype(jnp.bfloat16)

        for dev, par in PARENT.items():
            @pl.when(my == dev)
            def _(dev=dev, par=par):
                rd = pltpu.make_async_remote_copy(
                    src_ref=bcast, dst_ref=bcast,
                    send_sem=bsend_sems.at[0], recv_sem=brecv_sem,
                    device_id=(par,), device_id_type=pl.DeviceIdType.MESH)
                rd.wait_recv()

        for dev, childs in CHILDREN.items():
            @pl.when(my == dev)
            def _(dev=dev, childs=childs):
                descs = []
                for c in childs:
                    rd = pltpu.make_async_remote_copy(
                        src_ref=bcast, dst_ref=bcast,
                        send_sem=bsend_sems.at[c], recv_sem=brecv_sem,
                        device_id=(c,), device_id_type=pl.DeviceIdType.MESH)
                    rd.start()
                    descs.append(rd)
                for rd in descs:
                    rd.wait_send()

        @pl.when(my != 0)
        def _():
            out_ref[...] = bcast[...].astype(jnp.float32)

    out2 = pl.pallas_call(
        body,
        out_shape=jax.ShapeDtypeStruct((SQ, DM), jnp.float32),
        in_specs=[pl.BlockSpec(memory_space=pltpu.VMEM)] * 5,
        out_specs=pl.BlockSpec(memory_space=pltpu.VMEM),
        scratch_shapes=[
            pltpu.VMEM((SQ, DM), jnp.bfloat16),
            pltpu.VMEM((SQ, SKV_SHARD), jnp.float32),
            pltpu.VMEM((SQ, DM), jnp.float32),
            pltpu.VMEM((SQ, 2 * HQ), jnp.float32),
            pltpu.VMEM((N_SHARED, DM), jnp.float32),
            pltpu.VMEM((N_SHARED, 2 * HQ), jnp.float32),
            pltpu.VMEM((N_DEV, N_SHARED, DM), jnp.float32),
            pltpu.VMEM((N_DEV, N_SHARED, 2 * HQ), jnp.float32),
            pltpu.VMEM((SQ, DM), jnp.bfloat16),
            pltpu.VMEM((SQ, DM), jnp.bfloat16),
            pltpu.SemaphoreType.DMA((2,)),
            pltpu.SemaphoreType.DMA((N_DEV,)),
            pltpu.SemaphoreType.DMA((N_DEV,)),
            pltpu.SemaphoreType.DMA((N_DEV,)),
            pltpu.SemaphoreType.DMA,
        ],
        compiler_params=pltpu.CompilerParams(collective_id=0),
    )(x2, Wq, K2, V2, Wo)
    return out2.reshape(1, SQ, DM)


# baseline (device time: 135039 ns/iter reference)
import jax
import jax.numpy as jnp
from jax import lax
from jax.experimental import pallas as pl
from jax.experimental.pallas import tpu as pltpu

N_DEV = 8
SQ = 1024
SKV_SHARD = 1024
HQ = 8
DH = 128
DM = HQ * DH
SCALE = 0.08838834764831843
NEG = -1e9
GLOB_ROWS = 32
BAND_LO = 896
N_SHARED = GLOB_ROWS + (SQ - BAND_LO)

CHILDREN = {0: (4, 3, 1), 4: (7, 5), 3: (2,), 7: (6,)}
PARENT = {1: 0, 3: 0, 4: 0, 5: 4, 7: 4, 2: 3, 6: 7}


def kernel(x, Wq, K_ext, V_ext, Wo):
    x2 = x.reshape(SQ, DM)
    K2 = K_ext.reshape(SKV_SHARD, DM)
    V2 = V_ext.reshape(SKV_SHARD, DM)

    RB = 256

    def body(x_ref, wq_ref, k_ref, v_ref, wo_ref, out_ref,
             qbuf, part_n, part_md, send_n, send_md,
             gath_n, gath_md, ctx_bf, bcast,
             gsend_sems, grecvn_sems, grecvmd_sems, bsend_sems, brecv_sem):
        my = lax.axis_index("i")

        barrier = pltpu.get_barrier_semaphore()
        for s in range(N_DEV):
            pl.semaphore_signal(barrier, inc=1, device_id=(s,),
                                device_id_type=pl.DeviceIdType.MESH)
        pl.semaphore_wait(barrier, N_DEV)

        def q_step(i, carry):
            r0 = i * RB
            q = jnp.dot(x_ref[pl.ds(r0, RB), :].astype(jnp.bfloat16),
                        wq_ref[...].astype(jnp.bfloat16),
                        preferred_element_type=jnp.float32)
            qbuf[pl.ds(r0, RB), :] = q.astype(jnp.bfloat16)
            return carry
        lax.fori_loop(0, SQ // RB, q_step, 0)

        def flash_step(rb, carry):
            r0 = rb * RB
            rows = r0 + lax.broadcasted_iota(jnp.int32, (RB, SKV_SHARD), 0)
            cols = lax.broadcasted_iota(jnp.int32, (RB, SKV_SHARD), 1)
            ki = cols + my * SKV_SHARD
            keep = (jnp.abs(rows - ki) <= 128) | (ki < 32) | (rows < 32)
            bias = jnp.where(keep, 0.0, NEG).astype(jnp.float32)
            for h in range(HQ):
                c0 = h * DH
                kh = k_ref[:, c0:c0 + DH].astype(jnp.bfloat16)
                s = lax.dot_general(qbuf[pl.ds(r0, RB), c0:c0 + DH], kh,
                                    (((1,), (1,)), ((), ())),
                                    preferred_element_type=jnp.float32)
                s = s * SCALE + bias
                m = jnp.max(s, axis=1, keepdims=True)
                w = jnp.exp(s - m)
                d = jnp.sum(w, axis=1, keepdims=True)
                nh = jnp.dot(w.astype(jnp.bfloat16),
                             v_ref[:, c0:c0 + DH].astype(jnp.bfloat16),
                             preferred_element_type=jnp.float32)
                part_n[pl.ds(r0, RB), c0:c0 + DH] = nh
                part_md[pl.ds(r0, RB), h:h + 1] = m
                part_md[pl.ds(r0, RB), HQ + h:HQ + h + 1] = d
            return carry
        lax.fori_loop(0, SQ // RB, flash_step, 0)

        send_n[0:GLOB_ROWS, :] = part_n[0:GLOB_ROWS, :]
        send_n[GLOB_ROWS:N_SHARED, :] = part_n[BAND_LO:SQ, :]
        send_md[0:GLOB_ROWS, :] = part_md[0:GLOB_ROWS, :]
        send_md[GLOB_ROWS:N_SHARED, :] = part_md[BAND_LO:SQ, :]

        @pl.when(my == 0)
        def _():
            gath_n[0] = send_n[...]
            gath_md[0] = send_md[...]

        for s_idx in range(1, N_DEV):
            @pl.when(my == s_idx)
            def _(s_idx=s_idx):
                r1 = pltpu.make_async_remote_copy(
                    src_ref=send_n, dst_ref=gath_n.at[s_idx],
                    send_sem=gsend_sems.at[0], recv_sem=grecvn_sems.at[s_idx],
                    device_id=(0,), device_id_type=pl.DeviceIdType.MESH)
                r2 = pltpu.make_async_remote_copy(
                    src_ref=send_md, dst_ref=gath_md.at[s_idx],
                    send_sem=gsend_sems.at[1], recv_sem=grecvmd_sems.at[s_idx],
                    device_id=(0,), device_id_type=pl.DeviceIdType.MESH)
                r1.start()
                r2.start()
                r1.wait_send()
                r2.wait_send()

        @pl.when(my == 0)
        def _():
            for s_idx in range(1, N_DEV):
                rn = pltpu.make_async_remote_copy(
                    src_ref=gath_n.at[s_idx], dst_ref=gath_n.at[s_idx],
                    send_sem=gsend_sems.at[0], recv_sem=grecvn_sems.at[s_idx],
                    device_id=(0,), device_id_type=pl.DeviceIdType.MESH)
                rn.wait_recv()
                rmd = pltpu.make_async_remote_copy(
                    src_ref=gath_md.at[s_idx], dst_ref=gath_md.at[s_idx],
                    send_sem=gsend_sems.at[1], recv_sem=grecvmd_sems.at[s_idx],
                    device_id=(0,), device_id_type=pl.DeviceIdType.MESH)
                rmd.wait_recv()

            def norm_step(rb, carry):
                r0 = rb * RB
                for h in range(HQ):
                    c0 = h * DH
                    ctx_bf[pl.ds(r0, RB), c0:c0 + DH] = (
                        part_n[pl.ds(r0, RB), c0:c0 + DH]
                        / part_md[pl.ds(r0, RB), HQ + h:HQ + h + 1]
                    ).astype(jnp.bfloat16)
                return carry
            lax.fori_loop(0, SQ // RB, norm_step, 0)

            m_all = gath_md[0, :, 0:HQ]
            for s_idx in range(1, N_DEV):
                m_all = jnp.maximum(m_all, gath_md[s_idx, :, 0:HQ])
            scales = []
            den = jnp.zeros((N_SHARED, HQ), jnp.float32)
            for s_idx in range(N_DEV):
                c = jnp.exp(gath_md[s_idx, :, 0:HQ] - m_all)
                scales.append(c)
                den = den + gath_md[s_idx, :, HQ:2 * HQ] * c
            for h in range(HQ):
                c0, c1 = h * DH, (h + 1) * DH
                num = gath_n[0, :, c0:c1] * scales[0][:, h:h + 1]
                for s_idx in range(1, N_DEV):
                    num = num + gath_n[s_idx, :, c0:c1] * scales[s_idx][:, h:h + 1]
                ch = (num / den[:, h:h + 1]).astype(jnp.bfloat16)
                ctx_bf[0:GLOB_ROWS, c0:c1] = ch[0:GLOB_ROWS, :]
                ctx_bf[BAND_LO:SQ, c0:c1] = ch[GLOB_ROWS:N_SHARED, :]

            def proj_step(i, carry):
                r0 = i * RB
                o = jnp.dot(ctx_bf[pl.ds(r0, RB), :],
                            wo_ref[...].astype(jnp.bfloat16),
                            preferred_element_type=jnp.float32)
                out_ref[pl.ds(r0, RB), :] = o
                bcast[pl.ds(r0, RB), :] = o.astype(jnp.bfloat16)
                return carry
            lax.fori_loop(0, SQ // RB, proj_step, 0)

        for dev, par in PARENT.items():
            @pl.when(my == dev)
            def _(dev=dev, par=par):
                rd = pltpu.make_async_remote_copy(
                    src_ref=bcast, dst_ref=bcast,
                    send_sem=bsend_sems.at[0], recv_sem=brecv_sem,
                    device_id=(par,), device_id_type=pl.DeviceIdType.MESH)
                rd.wait_recv()

        for dev, childs in CHILDREN.items():
            @pl.when(my == dev)
            def _(dev=dev, childs=childs):
                descs = []
                for c in childs:
                    rd = pltpu.make_async_remote_copy(
                        src_ref=bcast, dst_ref=bcast,
                        send_sem=bsend_sems.at[c], recv_sem=brecv_sem,
                        device_id=(c,), device_id_type=pl.DeviceIdType.MESH)
                    rd.start()
                    descs.append(rd)
                for rd in descs:
                    rd.wait_send()

        @pl.when(my != 0)
        def _():
            def cp_step(i, carry):
                r0 = i * RB
                out_ref[pl.ds(r0, RB), :] = bcast[pl.ds(r0, RB), :].astype(
                    jnp.float32)
                return carry
            lax.fori_loop(0, SQ // RB, cp_step, 0)

    out2 = pl.pallas_call(
        body,
        out_shape=jax.ShapeDtypeStruct((SQ, DM), jnp.float32),
        in_specs=[pl.BlockSpec(memory_space=pltpu.VMEM)] * 5,
        out_specs=pl.BlockSpec(memory_space=pltpu.VMEM),
        scratch_shapes=[
            pltpu.VMEM((SQ, DM), jnp.bfloat16),
            pltpu.VMEM((SQ, DM), jnp.float32),
            pltpu.VMEM((SQ, 2 * HQ), jnp.float32),
            pltpu.VMEM((N_SHARED, DM), jnp.float32),
            pltpu.VMEM((N_SHARED, 2 * HQ), jnp.float32),
            pltpu.VMEM((N_DEV, N_SHARED, DM), jnp.float32),
            pltpu.VMEM((N_DEV, N_SHARED, 2 * HQ), jnp.float32),
            pltpu.VMEM((SQ, DM), jnp.bfloat16),
            pltpu.VMEM((SQ, DM), jnp.bfloat16),
            pltpu.SemaphoreType.DMA((2,)),
            pltpu.SemaphoreType.DMA((N_DEV,)),
            pltpu.SemaphoreType.DMA((N_DEV,)),
            pltpu.SemaphoreType.DMA((N_DEV,)),
            pltpu.SemaphoreType.DMA,
        ],
        compiler_params=pltpu.CompilerParams(
            collective_id=0, vmem_limit_bytes=100 * 1024 * 1024
        ),
    )(x2, Wq, K2, V2, Wo)
    return out2.reshape(1, SQ, DM)


# device time: 115370 ns/iter; 1.1705x vs baseline; 1.1705x over previous
import jax
import jax.numpy as jnp
from jax import lax
from jax.experimental import pallas as pl
from jax.experimental.pallas import tpu as pltpu

N_DEV = 8
SQ = 1024
SKV_SHARD = 1024
HQ = 8
DH = 128
DM = HQ * DH
SCALE = 0.08838834764831843
NEG = -1e9
GLOB_ROWS = 32
BAND_LO = 896
N_SHARED = GLOB_ROWS + (SQ - BAND_LO)

CHILDREN = {0: (4, 3, 1), 4: (7, 5), 3: (2,), 7: (6,)}
PARENT = {1: 0, 3: 0, 4: 0, 5: 4, 7: 4, 2: 3, 6: 7}


def kernel(x, Wq, K_ext, V_ext, Wo):
    x2 = x.reshape(SQ, DM)
    K2 = K_ext.reshape(SKV_SHARD, DM)
    V2 = V_ext.reshape(SKV_SHARD, DM)

    RB = 256

    def body(x_ref, wq_ref, k_ref, v_ref, wo_ref, out_ref,
             qbuf, part_n, part_md, send_n, send_md,
             gath_n, gath_md, ctx_bf, bcast,
             gsend_sems, grecvn_sems, grecvmd_sems, bsend_sems, brecv_sems):
        my = lax.axis_index("i")

        barrier = pltpu.get_barrier_semaphore()
        for s in range(N_DEV):
            pl.semaphore_signal(barrier, inc=1, device_id=(s,),
                                device_id_type=pl.DeviceIdType.MESH)
        pl.semaphore_wait(barrier, N_DEV)

        def q_step(i, carry):
            r0 = i * RB
            q = jnp.dot(x_ref[pl.ds(r0, RB), :].astype(jnp.bfloat16),
                        wq_ref[...].astype(jnp.bfloat16),
                        preferred_element_type=jnp.float32)
            qbuf[pl.ds(r0, RB), :] = q.astype(jnp.bfloat16)
            return carry
        lax.fori_loop(0, SQ // RB, q_step, 0)

        def flash_step(rb, carry):
            r0 = rb * RB
            rows = r0 + lax.broadcasted_iota(jnp.int32, (RB, SKV_SHARD), 0)
            cols = lax.broadcasted_iota(jnp.int32, (RB, SKV_SHARD), 1)
            ki = cols + my * SKV_SHARD
            keep = (jnp.abs(rows - ki) <= 128) | (ki < 32) | (rows < 32)
            bias = jnp.where(keep, 0.0, NEG).astype(jnp.float32)
            for h in range(HQ):
                c0 = h * DH
                kh = k_ref[:, c0:c0 + DH].astype(jnp.bfloat16)
                s = lax.dot_general(qbuf[pl.ds(r0, RB), c0:c0 + DH], kh,
                                    (((1,), (1,)), ((), ())),
                                    preferred_element_type=jnp.float32)
                s = s * SCALE + bias
                m = jnp.max(s, axis=1, keepdims=True)
                w = jnp.exp(s - m)
                d = jnp.sum(w, axis=1, keepdims=True)
                nh = jnp.dot(w.astype(jnp.bfloat16),
                             v_ref[:, c0:c0 + DH].astype(jnp.bfloat16),
                             preferred_element_type=jnp.float32)
                part_n[pl.ds(r0, RB), c0:c0 + DH] = nh
                part_md[pl.ds(r0, RB), h:h + 1] = m
                part_md[pl.ds(r0, RB), HQ + h:HQ + h + 1] = d
            return carry
        lax.fori_loop(0, SQ // RB, flash_step, 0)

        send_n[0:GLOB_ROWS, :] = part_n[0:GLOB_ROWS, :]
        send_n[GLOB_ROWS:N_SHARED, :] = part_n[BAND_LO:SQ, :]
        send_md[0:GLOB_ROWS, :] = part_md[0:GLOB_ROWS, :]
        send_md[GLOB_ROWS:N_SHARED, :] = part_md[BAND_LO:SQ, :]

        @pl.when(my == 0)
        def _():
            gath_n[0] = send_n[...]
            gath_md[0] = send_md[...]

        for s_idx in range(1, N_DEV):
            @pl.when(my == s_idx)
            def _(s_idx=s_idx):
                r1 = pltpu.make_async_remote_copy(
                    src_ref=send_n, dst_ref=gath_n.at[s_idx],
                    send_sem=gsend_sems.at[0], recv_sem=grecvn_sems.at[s_idx],
                    device_id=(0,), device_id_type=pl.DeviceIdType.MESH)
                r2 = pltpu.make_async_remote_copy(
                    src_ref=send_md, dst_ref=gath_md.at[s_idx],
                    send_sem=gsend_sems.at[1], recv_sem=grecvmd_sems.at[s_idx],
                    device_id=(0,), device_id_type=pl.DeviceIdType.MESH)
                r1.start()
                r2.start()
                r1.wait_send()
                r2.wait_send()

        @pl.when(my == 0)
        def _():
            for s_idx in range(1, N_DEV):
                rn = pltpu.make_async_remote_copy(
                    src_ref=gath_n.at[s_idx], dst_ref=gath_n.at[s_idx],
                    send_sem=gsend_sems.at[0], recv_sem=grecvn_sems.at[s_idx],
                    device_id=(0,), device_id_type=pl.DeviceIdType.MESH)
                rn.wait_recv()
                rmd = pltpu.make_async_remote_copy(
                    src_ref=gath_md.at[s_idx], dst_ref=gath_md.at[s_idx],
                    send_sem=gsend_sems.at[1], recv_sem=grecvmd_sems.at[s_idx],
                    device_id=(0,), device_id_type=pl.DeviceIdType.MESH)
                rmd.wait_recv()

            def norm_step(rb, carry):
                r0 = rb * RB
                for h in range(HQ):
                    c0 = h * DH
                    ctx_bf[pl.ds(r0, RB), c0:c0 + DH] = (
                        part_n[pl.ds(r0, RB), c0:c0 + DH]
                        / part_md[pl.ds(r0, RB), HQ + h:HQ + h + 1]
                    ).astype(jnp.bfloat16)
                return carry
            lax.fori_loop(0, SQ // RB, norm_step, 0)

            m_all = gath_md[0, :, 0:HQ]
            for s_idx in range(1, N_DEV):
                m_all = jnp.maximum(m_all, gath_md[s_idx, :, 0:HQ])
            scales = []
            den = jnp.zeros((N_SHARED, HQ), jnp.float32)
            for s_idx in range(N_DEV):
                c = jnp.exp(gath_md[s_idx, :, 0:HQ] - m_all)
                scales.append(c)
                den = den + gath_md[s_idx, :, HQ:2 * HQ] * c
            for h in range(HQ):
                c0, c1 = h * DH, (h + 1) * DH
                num = gath_n[0, :, c0:c1] * scales[0][:, h:h + 1]
                for s_idx in range(1, N_DEV):
                    num = num + gath_n[s_idx, :, c0:c1] * scales[s_idx][:, h:h + 1]
                ch = (num / den[:, h:h + 1]).astype(jnp.bfloat16)
                ctx_bf[0:GLOB_ROWS, c0:c1] = ch[0:GLOB_ROWS, :]
                ctx_bf[BAND_LO:SQ, c0:c1] = ch[GLOB_ROWS:N_SHARED, :]

            send_descs = []
            for rb in range(SQ // RB):
                r0 = rb * RB
                o = jnp.dot(ctx_bf[r0:r0 + RB, :],
                            wo_ref[...].astype(jnp.bfloat16),
                            preferred_element_type=jnp.float32)
                out_ref[r0:r0 + RB, :] = o
                bcast[r0:r0 + RB, :] = o.astype(jnp.bfloat16)
                blk = bcast.at[pl.ds(r0, RB)]
                for c in CHILDREN[0]:
                    rd = pltpu.make_async_remote_copy(
                        src_ref=blk, dst_ref=blk,
                        send_sem=bsend_sems.at[c * (SQ // RB) + rb],
                        recv_sem=brecv_sems.at[rb],
                        device_id=(c,), device_id_type=pl.DeviceIdType.MESH)
                    rd.start()
                    send_descs.append(rd)
            for rd in send_descs:
                rd.wait_send()

        for dev in sorted(PARENT):
            par = PARENT[dev]
            childs = CHILDREN.get(dev, ())

            @pl.when(my == dev)
            def _(par=par, childs=childs):
                sends = []
                for rb in range(SQ // RB):
                    blk = bcast.at[pl.ds(rb * RB, RB)]
                    rcv = pltpu.make_async_remote_copy(
                        src_ref=blk, dst_ref=blk,
                        send_sem=bsend_sems.at[0], recv_sem=brecv_sems.at[rb],
                        device_id=(par,), device_id_type=pl.DeviceIdType.MESH)
                    rcv.wait_recv()
                    for c in childs:
                        sd = pltpu.make_async_remote_copy(
                            src_ref=blk, dst_ref=blk,
                            send_sem=bsend_sems.at[c * (SQ // RB) + rb],
                            recv_sem=brecv_sems.at[rb],
                            device_id=(c,), device_id_type=pl.DeviceIdType.MESH)
                        sd.start()
                        sends.append(sd)
                for sd in sends:
                    sd.wait_send()

        @pl.when(my != 0)
        def _():
            def cp_step(i, carry):
                r0 = i * RB
                out_ref[pl.ds(r0, RB), :] = bcast[pl.ds(r0, RB), :].astype(
                    jnp.float32)
                return carry
            lax.fori_loop(0, SQ // RB, cp_step, 0)

    out2 = pl.pallas_call(
        body,
        out_shape=jax.ShapeDtypeStruct((SQ, DM), jnp.float32),
        in_specs=[pl.BlockSpec(memory_space=pltpu.VMEM)] * 5,
        out_specs=pl.BlockSpec(memory_space=pltpu.VMEM),
        scratch_shapes=[
            pltpu.VMEM((SQ, DM), jnp.bfloat16),
            pltpu.VMEM((SQ, DM), jnp.float32),
            pltpu.VMEM((SQ, 2 * HQ), jnp.float32),
            pltpu.VMEM((N_SHARED, DM), jnp.float32),
            pltpu.VMEM((N_SHARED, 2 * HQ), jnp.float32),
            pltpu.VMEM((N_DEV, N_SHARED, DM), jnp.float32),
            pltpu.VMEM((N_DEV, N_SHARED, 2 * HQ), jnp.float32),
            pltpu.VMEM((SQ, DM), jnp.bfloat16),
            pltpu.VMEM((SQ, DM), jnp.bfloat16),
            pltpu.SemaphoreType.DMA((2,)),
            pltpu.SemaphoreType.DMA((N_DEV,)),
            pltpu.SemaphoreType.DMA((N_DEV,)),
            pltpu.SemaphoreType.DMA((N_DEV * 4,)),
            pltpu.SemaphoreType.DMA((4,)),
        ],
        compiler_params=pltpu.CompilerParams(
            collective_id=0, vmem_limit_bytes=100 * 1024 * 1024
        ),
    )(x2, Wq, K2, V2, Wo)
    return out2.reshape(1, SQ, DM)


# device time: 73975 ns/iter; 1.8255x vs baseline; 1.5596x over previous
import jax
import jax.numpy as jnp
from jax import lax
from jax.experimental import pallas as pl
from jax.experimental.pallas import tpu as pltpu

N_DEV = 8
SQ = 1024
SKV_SHARD = 1024
HQ = 8
DH = 128
DM = HQ * DH
SCALE = 0.08838834764831843
NEG = -1e9
GLOB_ROWS = 32
BAND_LO = 896
BAND_ROWS = SQ - BAND_LO
RB = 256
NB = SQ // RB

CHILDREN = {0: (4, 3, 1), 4: (7, 5), 3: (2,), 7: (6,)}
PARENT = {1: 0, 3: 0, 4: 0, 5: 4, 7: 4, 2: 3, 6: 7}


def kernel(x, Wq, K_ext, V_ext, Wo):
    x2 = x.reshape(SQ, DM)
    K2 = K_ext.reshape(SKV_SHARD, DM)
    V2 = V_ext.reshape(SKV_SHARD, DM)

    def body(x_ref, wq_ref, k_ref, v_ref, wo_ref, out_ref,
             qbuf, part_n, part_d, sgn, sbn, gathg_n, gathg_d,
             gathb_n, gathb_d, ctx_bf, bcast,
             gsend_sems, ggn_sems, ggd_sems, gb_sems, bsend_sems, brecv_sems):
        my = lax.axis_index("i")

        barrier = pltpu.get_barrier_semaphore()
        for s in range(N_DEV):
            pl.semaphore_signal(barrier, inc=1, device_id=(s,),
                                device_id_type=pl.DeviceIdType.MESH)
        pl.semaphore_wait(barrier, N_DEV)

        def q_step(i, carry):
            r0 = i * RB
            q = jnp.dot(x_ref[pl.ds(r0, RB), :].astype(jnp.bfloat16),
                        wq_ref[...].astype(jnp.bfloat16),
                        preferred_element_type=jnp.float32)
            qbuf[pl.ds(r0, RB), :] = (q * SCALE).astype(jnp.bfloat16)
            return carry
        lax.fori_loop(0, NB, q_step, 0)

        def flash_block(rb):
            r0 = rb * RB
            rows = r0 + lax.broadcasted_iota(jnp.int32, (RB, SKV_SHARD), 0)
            cols = lax.broadcasted_iota(jnp.int32, (RB, SKV_SHARD), 1)
            ki = cols + my * SKV_SHARD
            keep = (jnp.abs(rows - ki) <= 128) | (ki < 32) | (rows < 32)
            bias = jnp.where(keep, 0.0, NEG).astype(jnp.float32)
            for h in range(HQ):
                c0 = h * DH
                kh = k_ref[:, c0:c0 + DH].astype(jnp.bfloat16)
                s = lax.dot_general(qbuf[pl.ds(r0, RB), c0:c0 + DH], kh,
                                    (((1,), (1,)), ((), ())),
                                    preferred_element_type=jnp.float32)
                w = jnp.exp(s + bias)
                d = jnp.sum(w, axis=1, keepdims=True)
                nh = jnp.dot(w.astype(jnp.bfloat16),
                             v_ref[:, c0:c0 + DH].astype(jnp.bfloat16),
                             preferred_element_type=jnp.float32)
                part_n[pl.ds(r0, RB), c0:c0 + DH] = nh
                part_d[pl.ds(r0, RB), h:h + 1] = d

        lax.fori_loop(0, 2, lambda i, c: (flash_block(i * 3), c)[1], 0)

        @pl.when(my != 0)
        def _():
            sgn[...] = part_n[0:GLOB_ROWS, :].astype(jnp.bfloat16)

        for s_idx in range(1, N_DEV):
            @pl.when(my == s_idx)
            def _(s_idx=s_idx):
                rn = pltpu.make_async_remote_copy(
                    src_ref=sgn, dst_ref=gathg_n.at[s_idx],
                    send_sem=gsend_sems.at[0], recv_sem=ggn_sems.at[s_idx],
                    device_id=(0,), device_id_type=pl.DeviceIdType.MESH)
                rd = pltpu.make_async_remote_copy(
                    src_ref=part_d.at[pl.ds(0, GLOB_ROWS)],
                    dst_ref=gathg_d.at[s_idx],
                    send_sem=gsend_sems.at[1], recv_sem=ggd_sems.at[s_idx],
                    device_id=(0,), device_id_type=pl.DeviceIdType.MESH)
                rn.start()
                rd.start()
                rn.wait_send()
                rd.wait_send()

        @pl.when(my == 1)
        def _():
            sbn[...] = part_n[BAND_LO:SQ, :].astype(jnp.bfloat16)
            bn = pltpu.make_async_remote_copy(
                src_ref=sbn, dst_ref=gathb_n,
                send_sem=gsend_sems.at[2], recv_sem=gb_sems.at[0],
                device_id=(0,), device_id_type=pl.DeviceIdType.MESH)
            bd = pltpu.make_async_remote_copy(
                src_ref=part_d.at[pl.ds(BAND_LO, BAND_ROWS)], dst_ref=gathb_d,
                send_sem=gsend_sems.at[3], recv_sem=gb_sems.at[1],
                device_id=(0,), device_id_type=pl.DeviceIdType.MESH)
            bn.start()
            bd.start()
            bn.wait_send()
            bd.wait_send()

        lax.fori_loop(0, 2, lambda i, c: (flash_block(i + 1), c)[1], 0)

        @pl.when(my == 0)
        def _():
            def norm_step(rb, carry):
                r0 = rb * RB
                for h in range(HQ):
                    c0 = h * DH
                    ctx_bf[pl.ds(r0, RB), c0:c0 + DH] = (
                        part_n[pl.ds(r0, RB), c0:c0 + DH]
                        / part_d[pl.ds(r0, RB), h:h + 1]
                    ).astype(jnp.bfloat16)
                return carry
            lax.fori_loop(0, NB, norm_step, 0)

            for s_idx in range(1, N_DEV):
                pltpu.make_async_remote_copy(
                    src_ref=gathg_n.at[s_idx], dst_ref=gathg_n.at[s_idx],
                    send_sem=gsend_sems.at[0], recv_sem=ggn_sems.at[s_idx],
                    device_id=(0,), device_id_type=pl.DeviceIdType.MESH
                ).wait_recv()
                pltpu.make_async_remote_copy(
                    src_ref=gathg_d.at[s_idx], dst_ref=gathg_d.at[s_idx],
                    send_sem=gsend_sems.at[1], recv_sem=ggd_sems.at[s_idx],
                    device_id=(0,), device_id_type=pl.DeviceIdType.MESH
                ).wait_recv()
            pltpu.make_async_remote_copy(
                src_ref=gathb_n, dst_ref=gathb_n,
                send_sem=gsend_sems.at[2], recv_sem=gb_sems.at[0],
                device_id=(0,), device_id_type=pl.DeviceIdType.MESH
            ).wait_recv()
            pltpu.make_async_remote_copy(
                src_ref=gathb_d, dst_ref=gathb_d,
                send_sem=gsend_sems.at[3], recv_sem=gb_sems.at[1],
                device_id=(0,), device_id_type=pl.DeviceIdType.MESH
            ).wait_recv()

            ng = part_n[0:GLOB_ROWS, :]
            dg = part_d[0:GLOB_ROWS, :]
            for s_idx in range(1, N_DEV):
                ng = ng + gathg_n[s_idx].astype(jnp.float32)
                dg = dg + gathg_d[s_idx]
            nb = part_n[BAND_LO:SQ, :] + gathb_n[...].astype(jnp.float32)
            db = part_d[BAND_LO:SQ, :] + gathb_d[...]
            for h in range(HQ):
                c0 = h * DH
                ctx_bf[0:GLOB_ROWS, c0:c0 + DH] = (
                    ng[:, c0:c0 + DH] / dg[:, h:h + 1]).astype(jnp.bfloat16)
                ctx_bf[BAND_LO:SQ, c0:c0 + DH] = (
                    nb[:, c0:c0 + DH] / db[:, h:h + 1]).astype(jnp.bfloat16)

            descs = []
            for rb in range(NB):
                r0 = rb * RB
                o = jnp.dot(ctx_bf[r0:r0 + RB, :],
                            wo_ref[...].astype(jnp.bfloat16),
                            preferred_element_type=jnp.float32)
                out_ref[r0:r0 + RB, :] = o
                bcast[r0:r0 + RB, :] = o.astype(jnp.bfloat16)
                blk = bcast.at[pl.ds(r0, RB)]
                for c in CHILDREN[0]:
                    rd = pltpu.make_async_remote_copy(
                        src_ref=blk, dst_ref=blk,
                        send_sem=bsend_sems.at[c * NB + rb],
                        recv_sem=brecv_sems.at[rb],
                        device_id=(c,), device_id_type=pl.DeviceIdType.MESH)
                    rd.start()
                    descs.append(rd)
            for rd in descs:
                rd.wait_send()

        for dev in sorted(PARENT):
            par = PARENT[dev]
            childs = CHILDREN.get(dev, ())

            @pl.when(my == dev)
            def _(par=par, childs=childs):
                descs = []
                for rb in range(NB):
                    blk = bcast.at[pl.ds(rb * RB, RB)]
                    pltpu.make_async_remote_copy(
                        src_ref=blk, dst_ref=blk,
                        send_sem=bsend_sems.at[0], recv_sem=brecv_sems.at[rb],
                        device_id=(par,), device_id_type=pl.DeviceIdType.MESH
                    ).wait_recv()
                    for c in childs:
                        sd = pltpu.make_async_remote_copy(
                            src_ref=blk, dst_ref=blk,
                            send_sem=bsend_sems.at[c * NB + rb],
                            recv_sem=brecv_sems.at[rb],
                            device_id=(c,), device_id_type=pl.DeviceIdType.MESH)
                        sd.start()
                        descs.append(sd)
                for sd in descs:
                    sd.wait_send()

        @pl.when(my != 0)
        def _():
            def cp_step(i, carry):
                r0 = i * RB
                out_ref[pl.ds(r0, RB), :] = bcast[pl.ds(r0, RB), :].astype(
                    jnp.float32)
                return carry
            lax.fori_loop(0, NB, cp_step, 0)

    out2 = pl.pallas_call(
        body,
        out_shape=jax.ShapeDtypeStruct((SQ, DM), jnp.float32),
        in_specs=[pl.BlockSpec(memory_space=pltpu.VMEM)] * 5,
        out_specs=pl.BlockSpec(memory_space=pltpu.VMEM),
        scratch_shapes=[
            pltpu.VMEM((SQ, DM), jnp.bfloat16),
            pltpu.VMEM((SQ, DM), jnp.float32),
            pltpu.VMEM((SQ, HQ), jnp.float32),
            pltpu.VMEM((GLOB_ROWS, DM), jnp.bfloat16),
            pltpu.VMEM((BAND_ROWS, DM), jnp.bfloat16),
            pltpu.VMEM((N_DEV, GLOB_ROWS, DM), jnp.bfloat16),
            pltpu.VMEM((N_DEV, GLOB_ROWS, HQ), jnp.float32),
            pltpu.VMEM((BAND_ROWS, DM), jnp.bfloat16),
            pltpu.VMEM((BAND_ROWS, HQ), jnp.float32),
            pltpu.VMEM((SQ, DM), jnp.bfloat16),
            pltpu.VMEM((SQ, DM), jnp.bfloat16),
            pltpu.SemaphoreType.DMA((4,)),
            pltpu.SemaphoreType.DMA((N_DEV,)),
            pltpu.SemaphoreType.DMA((N_DEV,)),
            pltpu.SemaphoreType.DMA((2,)),
            pltpu.SemaphoreType.DMA((N_DEV * NB,)),
            pltpu.SemaphoreType.DMA((NB,)),
        ],
        compiler_params=pltpu.CompilerParams(
            collective_id=0, vmem_limit_bytes=100 * 1024 * 1024
        ),
    )(x2, Wq, K2, V2, Wo)
    return out2.reshape(1, SQ, DM)


# device time: 56862 ns/iter; 2.3749x vs baseline; 1.3010x over previous
import jax
import jax.numpy as jnp
from jax import lax
from jax.experimental import pallas as pl
from jax.experimental.pallas import tpu as pltpu

N_DEV = 8
SQ = 1024
SKV_SHARD = 1024
HQ = 8
DH = 128
DM = HQ * DH
SCALE = 0.08838834764831843
NEG = -1e9
GLOB_ROWS = 32
BAND_LO = 896
BAND_ROWS = SQ - BAND_LO
RB = 256
NB = SQ // RB
BB = 128
NBB = SQ // BB
BORDER = (2, 3, 4, 5, 0, 1, 6, 7)

CHILDREN = {0: (4, 3, 1), 4: (7, 5), 3: (2,), 7: (6,)}
PARENT = {1: 0, 3: 0, 4: 0, 5: 4, 7: 4, 2: 3, 6: 7}


def kernel(x, Wq, K_ext, V_ext, Wo):
    x2 = x.reshape(SQ, DM)

    def body(x_ref, wq_ref, k_hbm, v_hbm, wo_ref, out_ref,
             qbuf, part_n, part_d, sgn, sbn, gathg_n, gathg_d,
             gathb_n, gathb_d, ctx_bf, bcast, k_vm, v_vm,
             gsend_sems, ggn_sems, ggd_sems, gb_sems, bsend_sems, brecv_sems,
             kv_sems):
        my = lax.axis_index("i")

        kv_dmas = []
        for h in range(HQ):
            kd = pltpu.make_async_copy(
                k_hbm.at[0, :, h, :], k_vm.at[h], kv_sems.at[h])
            vd = pltpu.make_async_copy(
                v_hbm.at[0, :, h, :], v_vm.at[h], kv_sems.at[HQ + h])
            kd.start()
            vd.start()
            kv_dmas.append(kd)
            kv_dmas.append(vd)

        barrier = pltpu.get_barrier_semaphore()
        for s in range(N_DEV):
            pl.semaphore_signal(barrier, inc=1, device_id=(s,),
                                device_id_type=pl.DeviceIdType.MESH)
        pl.semaphore_wait(barrier, N_DEV)

        def q_step(i, carry):
            r0 = i * RB
            q = jnp.dot(x_ref[pl.ds(r0, RB), :].astype(jnp.bfloat16),
                        wq_ref[...].astype(jnp.bfloat16),
                        preferred_element_type=jnp.float32)
            qbuf[pl.ds(r0, RB), :] = (q * SCALE).astype(jnp.bfloat16)
            return carry
        lax.fori_loop(0, NB, q_step, 0)

        WIN = 512
        HB = 128

        def flash_piece(r0, k0, nk, acc_n, acc_d):
            rows = r0 + lax.broadcasted_iota(jnp.int32, (RB, nk), 0)
            cols = lax.broadcasted_iota(jnp.int32, (RB, nk), 1) + k0
            ki = cols + my * SKV_SHARD
            keep = (jnp.abs(rows - ki) <= 128) | (ki < 32) | (rows < 32)
            bias = jnp.where(keep, 0.0, NEG).astype(jnp.float32)
            out_n, out_d = [], []
            for h in range(HQ):
                c0 = h * DH
                kh = k_vm[h, pl.ds(k0, nk), :].astype(jnp.bfloat16)
                s = lax.dot_general(qbuf[pl.ds(r0, RB), c0:c0 + DH], kh,
                                    (((1,), (1,)), ((), ())),
                                    preferred_element_type=jnp.float32)
                w = jnp.exp(s + bias)
                d = jnp.sum(w, axis=1, keepdims=True)
                nh = jnp.dot(w.astype(jnp.bfloat16),
                             v_vm[h, pl.ds(k0, nk), :].astype(jnp.bfloat16),
                             preferred_element_type=jnp.float32)
                out_n.append(nh if acc_n is None else acc_n[h] + nh)
                out_d.append(d if acc_d is None else acc_d[h] + d)
            return out_n, out_d

        def flash_block(r0, w0, with_head):
            ns, ds = flash_piece(r0, w0, WIN, None, None)
            if with_head:
                ns, ds = flash_piece(r0, 0, HB, ns, ds)
            for h in range(HQ):
                c0 = h * DH
                part_n[pl.ds(r0, RB), c0:c0 + DH] = ns[h]
                part_d[pl.ds(r0, RB), h:h + 1] = ds[h]

        for dma in kv_dmas:
            dma.wait()

        flash_block(0, 0, False)

        for h in range(HQ):
            c0 = h * DH
            kh = k_vm[h, WIN:SKV_SHARD, :].astype(jnp.bfloat16)
            s = lax.dot_general(qbuf[0:GLOB_ROWS, c0:c0 + DH], kh,
                                (((1,), (1,)), ((), ())),
                                preferred_element_type=jnp.float32)
            w = jnp.exp(s)
            d = jnp.sum(w, axis=1, keepdims=True)
            nh = jnp.dot(w.astype(jnp.bfloat16),
                         v_vm[h, WIN:SKV_SHARD, :].astype(jnp.bfloat16),
                         preferred_element_type=jnp.float32)
            part_n[0:GLOB_ROWS, c0:c0 + DH] = part_n[0:GLOB_ROWS, c0:c0 + DH] + nh
            part_d[0:GLOB_ROWS, h:h + 1] = part_d[0:GLOB_ROWS, h:h + 1] + d

        flash_block(3 * RB, SKV_SHARD - WIN, True)

        @pl.when(my != 0)
        def _():
            sgn[...] = part_n[0:GLOB_ROWS, :].astype(jnp.bfloat16)

        for s_idx in range(1, N_DEV):
            @pl.when(my == s_idx)
            def _(s_idx=s_idx):
                rn = pltpu.make_async_remote_copy(
                    src_ref=sgn, dst_ref=gathg_n.at[s_idx],
                    send_sem=gsend_sems.at[0], recv_sem=ggn_sems.at[s_idx],
                    device_id=(0,), device_id_type=pl.DeviceIdType.MESH)
                rd = pltpu.make_async_remote_copy(
                    src_ref=part_d.at[pl.ds(0, GLOB_ROWS)],
                    dst_ref=gathg_d.at[s_idx],
                    send_sem=gsend_sems.at[1], recv_sem=ggd_sems.at[s_idx],
                    device_id=(0,), device_id_type=pl.DeviceIdType.MESH)
                rn.start()
                rd.start()
                rn.wait_send()
                rd.wait_send()

        @pl.when(my == 1)
        def _():
            sbn[...] = part_n[BAND_LO:SQ, :].astype(jnp.bfloat16)
            bn = pltpu.make_async_remote_copy(
                src_ref=sbn, dst_ref=gathb_n,
                send_sem=gsend_sems.at[2], recv_sem=gb_sems.at[0],
                device_id=(0,), device_id_type=pl.DeviceIdType.MESH)
            bd = pltpu.make_async_remote_copy(
                src_ref=part_d.at[pl.ds(BAND_LO, BAND_ROWS)], dst_ref=gathb_d,
                send_sem=gsend_sems.at[3], recv_sem=gb_sems.at[1],
                device_id=(0,), device_id_type=pl.DeviceIdType.MESH)
            bn.start()
            bd.start()
            bn.wait_send()
            bd.wait_send()

        def mid_step(i, carry):
            rb = i + 1
            flash_block(rb * RB, rb * RB - HB, True)
            return carry
        lax.fori_loop(0, 2, mid_step, 0)

        @pl.when(my == 0)
        def _():
            descs = []

            def norm_block(rb):
                r0 = rb * RB
                for h in range(HQ):
                    c0 = h * DH
                    ctx_bf[r0:r0 + RB, c0:c0 + DH] = (
                        part_n[r0:r0 + RB, c0:c0 + DH]
                        / part_d[r0:r0 + RB, h:h + 1]
                    ).astype(jnp.bfloat16)

            def proj_block(bb):
                r0 = bb * BB
                o = jnp.dot(ctx_bf[r0:r0 + BB, :],
                            wo_ref[...].astype(jnp.bfloat16),
                            preferred_element_type=jnp.float32)
                out_ref[0, r0:r0 + BB, :] = o
                bcast[r0:r0 + BB, :] = o.astype(jnp.bfloat16)
                blk = bcast.at[pl.ds(r0, BB)]
                for c in CHILDREN[0]:
                    rd = pltpu.make_async_remote_copy(
                        src_ref=blk, dst_ref=blk,
                        send_sem=bsend_sems.at[c * NBB + bb],
                        recv_sem=brecv_sems.at[bb],
                        device_id=(c,), device_id_type=pl.DeviceIdType.MESH)
                    rd.start()
                    descs.append(rd)

            for rb in (1, 2):
                norm_block(rb)
                proj_block(2 * rb)
                proj_block(2 * rb + 1)

            for s_idx in range(1, N_DEV):
                pltpu.make_async_remote_copy(
                    src_ref=gathg_n.at[s_idx], dst_ref=gathg_n.at[s_idx],
                    send_sem=gsend_sems.at[0], recv_sem=ggn_sems.at[s_idx],
                    device_id=(0,), device_id_type=pl.DeviceIdType.MESH
                ).wait_recv()
                pltpu.make_async_remote_copy(
                    src_ref=gathg_d.at[s_idx], dst_ref=gathg_d.at[s_idx],
                    send_sem=gsend_sems.at[1], recv_sem=ggd_sems.at[s_idx],
                    device_id=(0,), device_id_type=pl.DeviceIdType.MESH
                ).wait_recv()
            pltpu.make_async_remote_copy(
                src_ref=gathb_n, dst_ref=gathb_n,
                send_sem=gsend_sems.at[2], recv_sem=gb_sems.at[0],
                device_id=(0,), device_id_type=pl.DeviceIdType.MESH
            ).wait_recv()
            pltpu.make_async_remote_copy(
                src_ref=gathb_d, dst_ref=gathb_d,
                send_sem=gsend_sems.at[3], recv_sem=gb_sems.at[1],
                device_id=(0,), device_id_type=pl.DeviceIdType.MESH
            ).wait_recv()

            norm_block(0)
            norm_block(3)
            ng = part_n[0:GLOB_ROWS, :]
            dg = part_d[0:GLOB_ROWS, :]
            for s_idx in range(1, N_DEV):
                ng = ng + gathg_n[s_idx].astype(jnp.float32)
                dg = dg + gathg_d[s_idx]
            nb = part_n[BAND_LO:SQ, :] + gathb_n[...].astype(jnp.float32)
            db = part_d[BAND_LO:SQ, :] + gathb_d[...]
            for h in range(HQ):
                c0 = h * DH
                ctx_bf[0:GLOB_ROWS, c0:c0 + DH] = (
                    ng[:, c0:c0 + DH] / dg[:, h:h + 1]).astype(jnp.bfloat16)
                ctx_bf[BAND_LO:SQ, c0:c0 + DH] = (
                    nb[:, c0:c0 + DH] / db[:, h:h + 1]).astype(jnp.bfloat16)

            proj_block(0)
            proj_block(1)
            proj_block(6)
            proj_block(7)
            for rd in descs:
                rd.wait_send()

        for dev in sorted(PARENT):
            par = PARENT[dev]
            childs = CHILDREN.get(dev, ())

            @pl.when(my == dev)
            def _(par=par, childs=childs):
                descs = []
                for bb in BORDER:
                    r0 = bb * BB
                    blk = bcast.at[pl.ds(r0, BB)]
                    pltpu.make_async_remote_copy(
                        src_ref=blk, dst_ref=blk,
                        send_sem=bsend_sems.at[0], recv_sem=brecv_sems.at[bb],
                        device_id=(par,), device_id_type=pl.DeviceIdType.MESH
                    ).wait_recv()
                    for c in childs:
                        sd = pltpu.make_async_remote_copy(
                            src_ref=blk, dst_ref=blk,
                            send_sem=bsend_sems.at[c * NBB + bb],
                            recv_sem=brecv_sems.at[bb],
                            device_id=(c,), device_id_type=pl.DeviceIdType.MESH)
                        sd.start()
                        descs.append(sd)
                    out_ref[0, r0:r0 + BB, :] = bcast[r0:r0 + BB, :].astype(
                        jnp.float32)
                for sd in descs:
                    sd.wait_send()

    return pl.pallas_call(
        body,
        out_shape=jax.ShapeDtypeStruct((1, SQ, DM), jnp.float32),
        in_specs=[
            pl.BlockSpec(memory_space=pltpu.VMEM),
            pl.BlockSpec(memory_space=pltpu.VMEM),
            pl.BlockSpec(memory_space=pltpu.MemorySpace.HBM),
            pl.BlockSpec(memory_space=pltpu.MemorySpace.HBM),
            pl.BlockSpec(memory_space=pltpu.VMEM),
        ],
        out_specs=pl.BlockSpec(memory_space=pltpu.VMEM),
        scratch_shapes=[
            pltpu.VMEM((SQ, DM), jnp.bfloat16),
            pltpu.VMEM((SQ, DM), jnp.float32),
            pltpu.VMEM((SQ, HQ), jnp.float32),
            pltpu.VMEM((GLOB_ROWS, DM), jnp.bfloat16),
            pltpu.VMEM((BAND_ROWS, DM), jnp.bfloat16),
            pltpu.VMEM((N_DEV, GLOB_ROWS, DM), jnp.bfloat16),
            pltpu.VMEM((N_DEV, GLOB_ROWS, HQ), jnp.float32),
            pltpu.VMEM((BAND_ROWS, DM), jnp.bfloat16),
            pltpu.VMEM((BAND_ROWS, HQ), jnp.float32),
            pltpu.VMEM((SQ, DM), jnp.bfloat16),
            pltpu.VMEM((SQ, DM), jnp.bfloat16),
            pltpu.VMEM((HQ, SKV_SHARD, DH), jnp.float32),
            pltpu.VMEM((HQ, SKV_SHARD, DH), jnp.float32),
            pltpu.SemaphoreType.DMA((4,)),
            pltpu.SemaphoreType.DMA((N_DEV,)),
            pltpu.SemaphoreType.DMA((N_DEV,)),
            pltpu.SemaphoreType.DMA((2,)),
            pltpu.SemaphoreType.DMA((N_DEV * NBB,)),
            pltpu.SemaphoreType.DMA((NBB,)),
            pltpu.SemaphoreType.DMA((2 * HQ,)),
        ],
        compiler_params=pltpu.CompilerParams(
            collective_id=0, vmem_limit_bytes=100 * 1024 * 1024
        ),
    )(x2, Wq, K_ext, V_ext, Wo)
